# baseline (speedup 1.0000x reference)
import functools

import jax
import jax.numpy as jnp
from jax import lax
from jax.experimental import pallas as pl
from jax.experimental.pallas import tpu as pltpu
from jax.experimental.pallas import tpu_sc as plsc

N = 16384
DIM = 32
DEPTH = 20
NC = 2
NS = 16
NW = NC * NS
TOK_W = N // NW
T = 32
NCH = TOK_W // T
CH = T * DEPTH
GB = 128
L = 16


def _body(x_hbm, dir_hbm, lab_hbm, emb1_hbm, emb2_hbm,
          out_hbm, tgt_hbm,
          idx_v, proj_v, dir_v, lab_v, hvec_v, out_v, tgt_v,
          sem_p, sem_g0, sem_g1):
    sem_g = (sem_g0, sem_g1)
    wid = lax.axis_index("s") * NC + lax.axis_index("c")
    tok0 = wid * TOK_W
    iota = lax.iota(jnp.int32, L)

    pltpu.sync_copy(x_hbm.at[pl.ds(tok0, TOK_W)], idx_v)
    for j in range(TOK_W // GB):
        pltpu.async_copy(
            emb1_hbm.at[idx_v.at[pl.ds(j * GB, GB)]],
            proj_v.at[pl.ds(j * GB, GB)], sem_p)

    def fire_chunk(c, b):
        pltpu.sync_copy(dir_hbm.at[pl.ds(tok0 * DEPTH + c * CH, CH)],
                        dir_v.at[pl.ds(b * CH, CH)])
        for j in range(CH // GB):
            pltpu.async_copy(
                emb2_hbm.at[dir_v.at[pl.ds(b * CH + j * GB, GB)]],
                hvec_v.at[pl.ds(b * CH + j * GB, GB)], sem_g[b])

    def wait_chunk(b):
        for j in range(CH // GB):
            pltpu.make_async_copy(
                emb2_hbm.at[dir_v.at[pl.ds(b * CH + j * GB, GB)]],
                hvec_v.at[pl.ds(b * CH + j * GB, GB)], sem_g[b]).wait()

    for b in range(2):
        fire_chunk(jnp.int32(b), b)
    for j in range(TOK_W // GB):
        pltpu.make_async_copy(
            emb1_hbm.at[idx_v.at[pl.ds(j * GB, GB)]],
            proj_v.at[pl.ds(j * GB, GB)], sem_p).wait()

    def compute_chunk(c, b):
        for blk in range(T // L):
            rows = [b * CH + blk * L * DEPTH + iota * DEPTH + k
                    for k in range(DEPTH)]
            rowp = c * T + blk * L + iota

            def dstep(d, accs):
                dv = jnp.full((L,), d, jnp.int32)
                pv = plsc.load_gather(proj_v, [rowp, dv])
                return tuple(
                    accs[k] + pv * plsc.load_gather(hvec_v, [rows[k], dv])
                    for k in range(DEPTH))

            accs = lax.fori_loop(
                0, DIM, dstep, (jnp.zeros((L,), jnp.float32),) * DEPTH)
            for k in range(DEPTH):
                x = accs[k]
                o = 1.0 / (1.0 + jnp.exp(-x))
                labv = plsc.load_gather(lab_v, [rows[k]])
                mi = jnp.where(x >= 0.0, 1, 0)
                t = jnp.where(mi == labv, 1.0, 0.0)
                plsc.store_scatter(out_v, [rows[k]], o)
                plsc.store_scatter(tgt_v, [rows[k]], t)

    def step(i, carry):
        for b in range(2):
            c = 2 * i + b
            wait_chunk(b)
            pltpu.sync_copy(lab_hbm.at[pl.ds(tok0 * DEPTH + c * CH, CH)],
                            lab_v.at[pl.ds(b * CH, CH)])
            compute_chunk(c, b)
            pltpu.sync_copy(out_v.at[pl.ds(b * CH, CH)],
                            out_hbm.at[pl.ds(tok0 * DEPTH + c * CH, CH)])
            pltpu.sync_copy(tgt_v.at[pl.ds(b * CH, CH)],
                            tgt_hbm.at[pl.ds(tok0 * DEPTH + c * CH, CH)])

            @pl.when(c + 2 < NCH)
            def _():
                fire_chunk(c + 2, b)
        return carry

    lax.fori_loop(0, NCH // 2, step, 0)


@jax.jit
def _run(x, dirf, labf, emb1, emb2):
    mesh = plsc.VectorSubcoreMesh(core_axis_name="c", subcore_axis_name="s",
                                  num_cores=NC, num_subcores=NS)
    f = pl.kernel(
        _body,
        out_type=(jax.ShapeDtypeStruct((N * DEPTH,), jnp.float32),
                  jax.ShapeDtypeStruct((N * DEPTH,), jnp.float32)),
        mesh=mesh,
        compiler_params=pltpu.CompilerParams(needs_layout_passes=False,
                                             use_tc_tiling_on_sc=False),
        scratch_types=[
            pltpu.VMEM((TOK_W,), jnp.int32),
            pltpu.VMEM((TOK_W, DIM), jnp.float32),
            pltpu.VMEM((2 * CH,), jnp.int32),
            pltpu.VMEM((2 * CH,), jnp.int32),
            pltpu.VMEM((2 * CH, DIM), jnp.float32),
            pltpu.VMEM((2 * CH,), jnp.float32),
            pltpu.VMEM((2 * CH,), jnp.float32),
            pltpu.SemaphoreType.DMA,
            pltpu.SemaphoreType.DMA,
            pltpu.SemaphoreType.DMA,
        ],
    )
    return f(x, dirf, labf, emb1, emb2)


def kernel(inputs, label, emb1, emb2):
    n, p, _, depth = label.shape
    assert (n, p, depth) == (N, 1, DEPTH) and emb1.shape[1] == DIM
    x = inputs.astype(jnp.int32)
    dirf = label[:, 0, 0, :].astype(jnp.int32).reshape(N * DEPTH)
    labf = label[:, 0, 1, :].astype(jnp.int32).reshape(N * DEPTH)
    o, t = _run(x, dirf, labf, emb1, emb2)
    return o.reshape(N, 1, DEPTH), t.reshape(N, 1, DEPTH)


# speedup vs baseline: 1.0084x; 1.0084x over previous
import jax
import jax.numpy as jnp
from jax import lax
from jax.experimental import pallas as pl
from jax.experimental.pallas import tpu as pltpu
from jax.experimental.pallas import tpu_sc as plsc

N = 16384
DIM = 32
DEPTH = 20
LW = 2 * DEPTH
NC = 2
NS = 16
NW = NC * NS
TOK_W = N // NW
T = 32
NCH = TOK_W // T
CH = T * DEPTH
GB = 128
L = 16


def _body(x_hbm, lab_hbm, emb1_hbm, emb2_hbm,
          out_hbm, tgt_hbm,
          idx_v, proj_v, dir_v, lab_v, hvec_v, out_v, tgt_v,
          sem_p, sem_g0, sem_g1, sem_o0, sem_o1):
    sem_g = (sem_g0, sem_g1)
    sem_o = (sem_o0, sem_o1)
    wid = lax.axis_index("s") * NC + lax.axis_index("c")
    tok0 = wid * TOK_W
    iota = lax.iota(jnp.int32, L)

    pltpu.sync_copy(x_hbm.at[pl.ds(tok0, TOK_W)], idx_v)
    for j in range(TOK_W // GB):
        pltpu.async_copy(
            emb1_hbm.at[idx_v.at[pl.ds(j * GB, GB)]],
            proj_v.at[pl.ds(j * GB, GB)], sem_p)

    def fire_chunk(c, b):
        pltpu.sync_copy(lab_hbm.at[pl.ds((tok0 + c * T) * LW, T * LW)],
                        lab_v.at[pl.ds(b * T * LW, T * LW)])
        for blk in range(T // L):
            src = b * T * LW + (blk * L + iota) * LW
            dst = b * CH + blk * L * DEPTH + iota * DEPTH
            for k in range(DEPTH):
                v = plsc.load_gather(lab_v, [src + k])
                plsc.store_scatter(dir_v, [dst + k], v)
        for j in range(CH // GB):
            pltpu.async_copy(
                emb2_hbm.at[dir_v.at[pl.ds(b * CH + j * GB, GB)]],
                hvec_v.at[pl.ds(b * CH + j * GB, GB)], sem_g[b])

    def wait_chunk(b):
        for j in range(CH // GB):
            pltpu.make_async_copy(
                emb2_hbm.at[dir_v.at[pl.ds(b * CH + j * GB, GB)]],
                hvec_v.at[pl.ds(b * CH + j * GB, GB)], sem_g[b]).wait()

    for b in range(2):
        fire_chunk(jnp.int32(b), b)
    for j in range(TOK_W // GB):
        pltpu.make_async_copy(
            emb1_hbm.at[idx_v.at[pl.ds(j * GB, GB)]],
            proj_v.at[pl.ds(j * GB, GB)], sem_p).wait()

    def out_copies(c, b):
        return (
            pltpu.make_async_copy(
                out_v.at[pl.ds(b * CH, CH)],
                out_hbm.at[pl.ds(tok0 * DEPTH + c * CH, CH)], sem_o[b]),
            pltpu.make_async_copy(
                tgt_v.at[pl.ds(b * CH, CH)],
                tgt_hbm.at[pl.ds(tok0 * DEPTH + c * CH, CH)], sem_o[b]),
        )

    def compute_chunk(c, b):
        for blk in range(T // L):
            rows = [b * CH + blk * L * DEPTH + iota * DEPTH + k
                    for k in range(DEPTH)]
            labr = b * T * LW + (blk * L + iota) * LW + DEPTH
            rowp = c * T + blk * L + iota

            def dstep(d, accs):
                dv = jnp.full((L,), d, jnp.int32)
                pv = plsc.load_gather(proj_v, [rowp, dv])
                return tuple(
                    accs[k] + pv * plsc.load_gather(hvec_v, [rows[k], dv])
                    for k in range(DEPTH))

            accs = lax.fori_loop(
                0, DIM, dstep, (jnp.zeros((L,), jnp.float32),) * DEPTH)
            for k in range(DEPTH):
                x = accs[k]
                o = 1.0 / (1.0 + jnp.exp(-x))
                labv = plsc.load_gather(lab_v, [labr + k])
                mi = jnp.where(x >= 0.0, 1, 0)
                t = jnp.where(mi == labv, 1.0, 0.0)
                plsc.store_scatter(out_v, [rows[k]], o)
                plsc.store_scatter(tgt_v, [rows[k]], t)

    def step(i, carry):
        for b in range(2):
            c = 2 * i + b
            wait_chunk(b)

            @pl.when(c >= 2)
            def _():
                for cp in out_copies(c - 2, b):
                    cp.wait()

            compute_chunk(c, b)
            for cp in out_copies(c, b):
                cp.start()

            @pl.when(c + 2 < NCH)
            def _():
                fire_chunk(c + 2, b)
        return carry

    lax.fori_loop(0, NCH // 2, step, 0)
    for b in range(2):
        for cp in out_copies(NCH - 2 + b, b):
            cp.wait()


@jax.jit
def _run(x, labf, emb1, emb2):
    mesh = plsc.VectorSubcoreMesh(core_axis_name="c", subcore_axis_name="s",
                                  num_cores=NC, num_subcores=NS)
    f = pl.kernel(
        _body,
        out_type=(jax.ShapeDtypeStruct((N * DEPTH,), jnp.float32),
                  jax.ShapeDtypeStruct((N * DEPTH,), jnp.float32)),
        mesh=mesh,
        compiler_params=pltpu.CompilerParams(needs_layout_passes=False,
                                             use_tc_tiling_on_sc=False),
        scratch_types=[
            pltpu.VMEM((TOK_W,), jnp.int32),
            pltpu.VMEM((TOK_W, DIM), jnp.float32),
            pltpu.VMEM((2 * CH,), jnp.int32),
            pltpu.VMEM((2 * T * LW,), jnp.int32),
            pltpu.VMEM((2 * CH, DIM), jnp.float32),
            pltpu.VMEM((2 * CH,), jnp.float32),
            pltpu.VMEM((2 * CH,), jnp.float32),
            pltpu.SemaphoreType.DMA,
            pltpu.SemaphoreType.DMA,
            pltpu.SemaphoreType.DMA,
            pltpu.SemaphoreType.DMA,
            pltpu.SemaphoreType.DMA,
        ],
    )
    return f(x, labf, emb1, emb2)


def kernel(inputs, label, emb1, emb2):
    n, p, _, depth = label.shape
    assert (n, p, depth) == (N, 1, DEPTH) and emb1.shape[1] == DIM
    x = inputs.astype(jnp.int32)
    labf = label.astype(jnp.int32).reshape(N * LW)
    o, t = _run(x, labf, emb1, emb2)
    return o.reshape(N, 1, DEPTH), t.reshape(N, 1, DEPTH)
